# baseline (speedup 1.0000x reference)
import functools

import jax
import jax.numpy as jnp
from jax import lax
from jax.experimental import pallas as pl
from jax.experimental.pallas import tpu as pltpu
from jax.experimental.pallas import tpu_sc as plsc

NUM_CORES = 2
NUM_SUBCORES = 16
NUM_WORKERS = NUM_CORES * NUM_SUBCORES


def _pos_embed_body(batch, rows_per_worker,
                    w_hbm, sp_hbm, out_hbm, sp_v, sem):
    core = lax.axis_index("c")
    sub = lax.axis_index("s")
    wid = sub * NUM_CORES + core
    base = wid * rows_per_worker

    pltpu.sync_copy(sp_hbm, sp_v)
    start = pl.multiple_of(sp_v[...][0], 8)

    copies = []
    for b in range(batch):
        copies.append(pltpu.async_copy(
            w_hbm.at[pl.ds(start + base, rows_per_worker)],
            out_hbm.at[b, pl.ds(base, rows_per_worker)],
            sem))
    for cp in copies:
        cp.wait()


def kernel(tokens, start_pos, W_pos):
    batch, seq_len = tokens.shape
    d_model = W_pos.shape[-1]
    assert seq_len % NUM_WORKERS == 0
    rows_per_worker = seq_len // NUM_WORKERS

    sp_arr = jnp.full((16,), start_pos, dtype=jnp.int32)

    mesh = plsc.VectorSubcoreMesh(
        core_axis_name="c", subcore_axis_name="s",
        num_cores=NUM_CORES, num_subcores=NUM_SUBCORES)

    body = functools.partial(_pos_embed_body, batch, rows_per_worker)

    out = pl.kernel(
        body,
        out_type=jax.ShapeDtypeStruct((batch, seq_len, d_model), W_pos.dtype),
        mesh=mesh,
        scratch_types=[
            pltpu.VMEM((16,), jnp.int32),
            pltpu.SemaphoreType.DMA,
        ],
    )(W_pos, sp_arr)
    return out


# speedup vs baseline: 51.1864x; 51.1864x over previous
import functools

import jax
import jax.numpy as jnp
from jax import lax
from jax.experimental import pallas as pl
from jax.experimental.pallas import tpu as pltpu
from jax.experimental.pallas import tpu_sc as plsc

NUM_CORES = 2
NUM_SUBCORES = 16
NUM_WORKERS = NUM_CORES * NUM_SUBCORES

CHUNK_ROWS = 16
NBUF = 2


def _pos_embed_body(batch, chunks_per_worker,
                    w_hbm, sp_hbm, out_hbm,
                    sp_v, buf0, buf1, li0, li1, so0, so1):
    core = lax.axis_index("c")
    sub = lax.axis_index("s")
    wid = sub * NUM_CORES + core
    rows_per_worker = chunks_per_worker * CHUNK_ROWS
    base = wid * rows_per_worker

    bufs = [buf0, buf1]
    lsems = [li0, li1]
    ssems = [so0, so1]

    pltpu.sync_copy(sp_hbm, sp_v)
    start = pl.multiple_of(sp_v[...][0], 8)

    def load(c):
        return pltpu.async_copy(
            w_hbm.at[pl.ds(start + base + c * CHUNK_ROWS, CHUNK_ROWS)],
            bufs[c % NBUF], lsems[c % NBUF])

    def store(c):
        return [pltpu.async_copy(
            bufs[c % NBUF],
            out_hbm.at[b, pl.ds(base + c * CHUNK_ROWS, CHUNK_ROWS)],
            ssems[c % NBUF]) for b in range(batch)]

    loads = [None] * chunks_per_worker
    stores = [None] * chunks_per_worker
    loads[0] = load(0)
    for c in range(chunks_per_worker):
        if c + 1 < chunks_per_worker:
            if c - 1 >= 0:
                for cp in stores[c - 1]:
                    cp.wait()
            loads[c + 1] = load(c + 1)
        loads[c].wait()
        stores[c] = store(c)
    for c in (chunks_per_worker - 2, chunks_per_worker - 1):
        if c >= 0:
            for cp in stores[c]:
                cp.wait()


def kernel(tokens, start_pos, W_pos):
    batch, seq_len = tokens.shape
    d_model = W_pos.shape[-1]
    assert seq_len % (NUM_WORKERS * CHUNK_ROWS) == 0
    chunks_per_worker = seq_len // (NUM_WORKERS * CHUNK_ROWS)

    sp_arr = jnp.full((16,), start_pos, dtype=jnp.int32)

    mesh = plsc.VectorSubcoreMesh(
        core_axis_name="c", subcore_axis_name="s",
        num_cores=NUM_CORES, num_subcores=NUM_SUBCORES)

    body = functools.partial(_pos_embed_body, batch, chunks_per_worker)

    out = pl.kernel(
        body,
        out_type=jax.ShapeDtypeStruct((batch, seq_len, d_model), W_pos.dtype),
        mesh=mesh,
        scratch_types=[
            pltpu.VMEM((16,), jnp.int32),
            pltpu.VMEM((CHUNK_ROWS, d_model), W_pos.dtype),
            pltpu.VMEM((CHUNK_ROWS, d_model), W_pos.dtype),
            pltpu.SemaphoreType.DMA,
            pltpu.SemaphoreType.DMA,
            pltpu.SemaphoreType.DMA,
            pltpu.SemaphoreType.DMA,
        ],
    )(W_pos, sp_arr)
    return out


# speedup vs baseline: 76.3318x; 1.4913x over previous
import jax
import jax.numpy as jnp
from jax.experimental import pallas as pl
from jax.experimental.pallas import tpu as pltpu

S_BLK = 256


def _body(sref, w_ref, out_ref):
    out_ref[...] = jnp.broadcast_to(w_ref[...][None], out_ref.shape)


def kernel(tokens, start_pos, W_pos):
    batch, seq_len = tokens.shape
    d_model = W_pos.shape[-1]
    grid = (seq_len // S_BLK,)

    sp_arr = jnp.full((1,), start_pos, dtype=jnp.int32)

    out = pl.pallas_call(
        _body,
        grid_spec=pltpu.PrefetchScalarGridSpec(
            num_scalar_prefetch=1,
            grid=grid,
            in_specs=[pl.BlockSpec((S_BLK, d_model),
                                   lambda i, s: (s[0] // S_BLK + i, 0))],
            out_specs=pl.BlockSpec((batch, S_BLK, d_model),
                                   lambda i, s: (0, i, 0)),
        ),
        out_shape=jax.ShapeDtypeStruct((batch, seq_len, d_model), W_pos.dtype),
    )(sp_arr, W_pos)
    return out
